# baseline (speedup 1.0000x reference)
import functools

import jax
import jax.numpy as jnp
from jax.experimental import pallas as pl
from jax.experimental.pallas import tpu as pltpu

_N_TABLE_CHUNKS = 8


def _issue_tile(ids_ref, table_hbm, emb_buf, sems, base, slot, tm):
    for r in range(tm):
        idx = ids_ref[base + r]
        pltpu.make_async_copy(
            table_hbm.at[pl.ds(idx, 1), :],
            emb_buf.at[slot, pl.ds(r, 1), :],
            sems.at[slot],
        ).start()


def _body(ids_ref, table_hbm, table2_hbm, w_ref, b_ref, out_ref,
          emb_buf, sems, table_vmem, tsem, tile_buf, *, n_tiles, tm, k_dma):
    j = pl.program_id(0)
    V2 = table_vmem.shape[0]
    stride = tm + 8

    @pl.when(j == 0)
    def _start():
        if n_tiles > k_dma:
            rows = V2 // _N_TABLE_CHUNKS
            for c in range(_N_TABLE_CHUNKS):
                lo = c * rows
                n_rows = rows if c < _N_TABLE_CHUNKS - 1 else V2 - lo
                pltpu.make_async_copy(
                    table2_hbm.at[pl.ds(lo, n_rows), :],
                    table_vmem.at[pl.ds(lo, n_rows), :],
                    tsem,
                ).start()
        _issue_tile(ids_ref, table_hbm, emb_buf, sems, 0, 0, tm)

    @pl.when(j + 1 < k_dma)
    def _prefetch():
        nxt = jax.lax.rem(j + 1, 2)
        _issue_tile(ids_ref, table_hbm, emb_buf, sems, (j + 1) * tm, nxt, tm)

    @pl.when(j < k_dma)
    def _dma_tile():
        cur = jax.lax.rem(j, 2)
        pltpu.make_async_copy(
            table_hbm.at[pl.ds(0, tm), :], emb_buf.at[cur], sems.at[cur]
        ).wait()

        @pl.when(cur == 0)
        def _mm0():
            out_ref[...] = jnp.dot(emb_buf[0], w_ref[...],
                                   preferred_element_type=jnp.float32) + b_ref[...]

        @pl.when(cur == 1)
        def _mm1():
            out_ref[...] = jnp.dot(emb_buf[1], w_ref[...],
                                   preferred_element_type=jnp.float32) + b_ref[...]

    if n_tiles > k_dma:
        @pl.when(j == k_dma)
        def _table_ready():
            pltpu.make_async_copy(table2_hbm, table_vmem, tsem).wait()

        @pl.when(j >= k_dma)
        def _vld_tile():
            base = j * tm
            for r in range(tm):
                i2 = pl.multiple_of(ids_ref[base + r] * 2, 2)
                slab = table_vmem[pl.ds(i2, 2), :]
                tile_buf[r:r + 2 * stride:stride, :] = slab
            lo = tile_buf[pl.ds(0, tm), :]
            hi = tile_buf[pl.ds(stride, tm), :]
            out_ref[...] = (
                jnp.dot(lo, w_ref[pl.ds(0, 128), :],
                        preferred_element_type=jnp.float32)
                + jnp.dot(hi, w_ref[pl.ds(128, 128), :],
                          preferred_element_type=jnp.float32)
                + b_ref[...])


@functools.partial(jax.jit, static_argnames=("tm", "k_dma"))
def _forward(ids, table, w, b, *, tm=256, k_dma=14):
    B, S = ids.shape
    V, E = table.shape
    H = w.shape[1]
    N = B * S

    tm_eff = max(8, min(int(tm), ((N + 7) // 8) * 8))
    tm_eff = ((tm_eff + 7) // 8) * 8
    n_tiles = (N + tm_eff - 1) // tm_eff
    n_pad = n_tiles * tm_eff
    k_dma = min(int(k_dma), n_tiles)

    ids_flat = jnp.clip(ids.reshape(-1).astype(jnp.int32), 0, V - 1)
    if n_pad != N:
        ids_flat = jnp.pad(ids_flat, (0, n_pad - N))

    table2 = table.reshape(2 * V, E // 2)
    stride = tm_eff + 8

    out = pl.pallas_call(
        functools.partial(_body, n_tiles=n_tiles, tm=tm_eff, k_dma=k_dma),
        out_shape=jax.ShapeDtypeStruct((n_pad, H), jnp.float32),
        grid_spec=pltpu.PrefetchScalarGridSpec(
            num_scalar_prefetch=1,
            grid=(n_tiles,),
            in_specs=[
                pl.BlockSpec(memory_space=pl.ANY),
                pl.BlockSpec(memory_space=pl.ANY),
                pl.BlockSpec((E, H), lambda j, ids: (0, 0)),
                pl.BlockSpec((1, H), lambda j, ids: (0, 0)),
            ],
            out_specs=pl.BlockSpec((tm_eff, H), lambda j, ids: (j, 0)),
            scratch_shapes=[
                pltpu.VMEM((2, tm_eff, E), jnp.float32),
                pltpu.SemaphoreType.DMA((2,)),
                pltpu.VMEM((2 * V, E // 2), jnp.float32),
                pltpu.SemaphoreType.DMA(()),
                pltpu.VMEM((2 * stride, E // 2), jnp.float32),
            ],
        ),
        compiler_params=pltpu.CompilerParams(
            dimension_semantics=("arbitrary",),
            disable_bounds_checks=True,
        ),
    )(ids_flat, table, table2, w, b)
    return out[:N].reshape(B, S, H)


def kernel(ids, table, w, b):
    return _forward(ids, table, w, b, tm=256, k_dma=1)


# speedup vs baseline: 1.0082x; 1.0082x over previous
import functools

import jax
import jax.numpy as jnp
from jax.experimental import pallas as pl
from jax.experimental.pallas import tpu as pltpu

_N_TABLE_CHUNKS = 8


def _body(ids_ref, table2_hbm, w_ref, b_ref, out_ref,
          table_vmem, tsem, tile_buf, *, tm):
    j = pl.program_id(0)
    V2 = table_vmem.shape[0]
    stride = tm + 8

    @pl.when(j == 0)
    def _load_table():
        rows = V2 // _N_TABLE_CHUNKS
        for c in range(_N_TABLE_CHUNKS):
            lo = c * rows
            n_rows = rows if c < _N_TABLE_CHUNKS - 1 else V2 - lo
            pltpu.make_async_copy(
                table2_hbm.at[pl.ds(lo, n_rows), :],
                table_vmem.at[pl.ds(lo, n_rows), :],
                tsem,
            ).start()
        pltpu.make_async_copy(table2_hbm, table_vmem, tsem).wait()

    base = j * tm
    for r in range(tm):
        i2 = pl.multiple_of(ids_ref[base + r], 2)
        slab = table_vmem[pl.ds(i2, 2), :]
        tile_buf[r:r + 2 * stride:stride, :] = slab
    lo_chunk = tile_buf[pl.ds(0, tm), :]
    hi_chunk = tile_buf[pl.ds(stride, tm), :]
    out_ref[...] = (
        jnp.dot(lo_chunk, w_ref[pl.ds(0, 128), :],
                preferred_element_type=jnp.float32)
        + jnp.dot(hi_chunk, w_ref[pl.ds(128, 128), :],
                  preferred_element_type=jnp.float32)
        + b_ref[...])


@functools.partial(jax.jit, static_argnames=("tm",))
def _forward(ids, table, w, b, *, tm=256):
    B, S = ids.shape
    V, E = table.shape
    H = w.shape[1]
    N = B * S

    tm_eff = max(8, min(int(tm), ((N + 7) // 8) * 8))
    tm_eff = ((tm_eff + 7) // 8) * 8
    n_tiles = (N + tm_eff - 1) // tm_eff
    n_pad = n_tiles * tm_eff

    ids_flat = jnp.clip(ids.reshape(-1).astype(jnp.int32), 0, V - 1) * 2
    if n_pad != N:
        ids_flat = jnp.pad(ids_flat, (0, n_pad - N))

    table2 = table.reshape(2 * V, E // 2)
    stride = tm_eff + 8

    out = pl.pallas_call(
        functools.partial(_body, tm=tm_eff),
        out_shape=jax.ShapeDtypeStruct((n_pad, H), jnp.float32),
        grid_spec=pltpu.PrefetchScalarGridSpec(
            num_scalar_prefetch=1,
            grid=(n_tiles,),
            in_specs=[
                pl.BlockSpec(memory_space=pl.ANY),
                pl.BlockSpec((E, H), lambda j, ids: (0, 0)),
                pl.BlockSpec((1, H), lambda j, ids: (0, 0)),
            ],
            out_specs=pl.BlockSpec((tm_eff, H), lambda j, ids: (j, 0)),
            scratch_shapes=[
                pltpu.VMEM((2 * V, E // 2), jnp.float32),
                pltpu.SemaphoreType.DMA(()),
                pltpu.VMEM((2 * stride, E // 2), jnp.float32),
            ],
        ),
        compiler_params=pltpu.CompilerParams(
            dimension_semantics=("arbitrary",),
            disable_bounds_checks=True,
        ),
    )(ids_flat, table2, w, b)
    return out[:N].reshape(B, S, H)


def kernel(ids, table, w, b):
    return _forward(ids, table, w, b, tm=256)


# speedup vs baseline: 1.9565x; 1.9406x over previous
import functools

import jax
import jax.numpy as jnp
from jax.experimental import pallas as pl
from jax.experimental.pallas import tpu as pltpu


def _issue_tile(ids_ref, table_hbm, emb_buf, sems, base, slot, tm):
    for r in range(tm):
        idx = ids_ref[base + r]
        pltpu.make_async_copy(
            table_hbm.at[pl.ds(idx, 1), :],
            emb_buf.at[slot, pl.ds(r, 1), :],
            sems.at[slot],
        ).start(priority=r % 2)


def _embed_project_body(ids_ref, table_hbm, w_ref, b_ref, out_ref,
                        emb_buf, sems, *, n_inner, tm):
    j = pl.program_id(0)

    @pl.when(j == 0)
    def _prime():
        _issue_tile(ids_ref, table_hbm, emb_buf, sems, j * tm, 0, tm)

    @pl.when(j + 1 < n_inner)
    def _prefetch():
        nxt = jax.lax.rem(j + 1, 2)
        _issue_tile(ids_ref, table_hbm, emb_buf, sems, (j + 1) * tm, nxt, tm)

    cur = jax.lax.rem(j, 2)
    pltpu.make_async_copy(
        table_hbm.at[pl.ds(0, tm), :], emb_buf.at[cur], sems.at[cur]
    ).wait()

    @pl.when(cur == 0)
    def _mm0():
        out_ref[...] = jnp.dot(emb_buf[0], w_ref[...],
                               preferred_element_type=jnp.float32) + b_ref[...]

    @pl.when(cur == 1)
    def _mm1():
        out_ref[...] = jnp.dot(emb_buf[1], w_ref[...],
                               preferred_element_type=jnp.float32) + b_ref[...]


@functools.partial(jax.jit, static_argnames=("tm",))
def _forward(ids, table, w, b, *, tm=256):
    B, S = ids.shape
    V, E = table.shape
    H = w.shape[1]
    N = B * S

    tm_eff = max(8, min(int(tm), ((N + 7) // 8) * 8))
    tm_eff = ((tm_eff + 7) // 8) * 8
    n_tiles = (N + tm_eff - 1) // tm_eff
    n_pad = n_tiles * tm_eff

    ids_flat = jnp.clip(ids.reshape(-1).astype(jnp.int32), 0, V - 1)
    if n_pad != N:
        ids_flat = jnp.pad(ids_flat, (0, n_pad - N))

    out = pl.pallas_call(
        functools.partial(_embed_project_body, n_inner=n_tiles, tm=tm_eff),
        out_shape=jax.ShapeDtypeStruct((n_pad, H), jnp.float32),
        grid_spec=pltpu.PrefetchScalarGridSpec(
            num_scalar_prefetch=1,
            grid=(n_tiles,),
            in_specs=[
                pl.BlockSpec(memory_space=pl.ANY),
                pl.BlockSpec((E, H), lambda j, ids: (0, 0)),
                pl.BlockSpec((1, H), lambda j, ids: (0, 0)),
            ],
            out_specs=pl.BlockSpec((tm_eff, H), lambda j, ids: (j, 0)),
            scratch_shapes=[
                pltpu.VMEM((2, tm_eff, E), table.dtype),
                pltpu.SemaphoreType.DMA((2,)),
            ],
        ),
        compiler_params=pltpu.CompilerParams(
            dimension_semantics=("arbitrary",),
            disable_bounds_checks=True,
        ),
    )(ids_flat, table, w, b)
    return out[:N].reshape(B, S, H)


def kernel(ids, table, w, b):
    return _forward(ids, table, w, b, tm=256)


# speedup vs baseline: 2.1187x; 1.0829x over previous
import functools

import jax
import jax.numpy as jnp
from jax.experimental import pallas as pl
from jax.experimental.pallas import tpu as pltpu


def _issue_tile(ids_ref, table_hbm, emb_buf, sems, base, slot, tm):
    for r in range(tm):
        idx = ids_ref[base + r]
        pltpu.make_async_copy(
            table_hbm.at[pl.ds(idx, 1), :],
            emb_buf.at[slot, pl.ds(r, 1), :],
            sems.at[slot],
        ).start()


def _embed_project_body(ids_ref, table_hbm, w_ref, b_ref, out_ref,
                        emb_buf, sems, *, n_inner, tm):
    j = pl.program_id(0)

    @pl.when(j == 0)
    def _prime():
        _issue_tile(ids_ref, table_hbm, emb_buf, sems, j * tm, 0, tm)

    @pl.when(j + 1 < n_inner)
    def _prefetch():
        nxt = jax.lax.rem(j + 1, 2)
        _issue_tile(ids_ref, table_hbm, emb_buf, sems, (j + 1) * tm, nxt, tm)

    cur = jax.lax.rem(j, 2)
    pltpu.make_async_copy(
        table_hbm.at[pl.ds(0, tm), :], emb_buf.at[cur], sems.at[cur]
    ).wait()

    @pl.when(cur == 0)
    def _mm0():
        out_ref[...] = jnp.dot(emb_buf[0], w_ref[...],
                               preferred_element_type=jnp.float32) + b_ref[...]

    @pl.when(cur == 1)
    def _mm1():
        out_ref[...] = jnp.dot(emb_buf[1], w_ref[...],
                               preferred_element_type=jnp.float32) + b_ref[...]


@functools.partial(jax.jit, static_argnames=("tm",))
def _forward(ids, table, w, b, *, tm=256):
    B, S = ids.shape
    V, E = table.shape
    H = w.shape[1]
    N = B * S

    tm_eff = max(8, min(int(tm), ((N + 7) // 8) * 8))
    tm_eff = ((tm_eff + 7) // 8) * 8
    n_tiles = (N + tm_eff - 1) // tm_eff
    n_pad = n_tiles * tm_eff

    ids_flat = jnp.clip(ids.reshape(-1).astype(jnp.int32), 0, V - 1)
    if n_pad != N:
        ids_flat = jnp.pad(ids_flat, (0, n_pad - N))

    out = pl.pallas_call(
        functools.partial(_embed_project_body, n_inner=n_tiles, tm=tm_eff),
        out_shape=jax.ShapeDtypeStruct((n_pad, H), jnp.float32),
        grid_spec=pltpu.PrefetchScalarGridSpec(
            num_scalar_prefetch=1,
            grid=(n_tiles,),
            in_specs=[
                pl.BlockSpec(memory_space=pl.ANY),
                pl.BlockSpec((E, H), lambda j, ids: (0, 0)),
                pl.BlockSpec((1, H), lambda j, ids: (0, 0)),
            ],
            out_specs=pl.BlockSpec((tm_eff, H), lambda j, ids: (j, 0)),
            scratch_shapes=[
                pltpu.VMEM((2, tm_eff, E), table.dtype),
                pltpu.SemaphoreType.DMA((2,)),
            ],
        ),
        compiler_params=pltpu.CompilerParams(
            dimension_semantics=("arbitrary",),
            disable_bounds_checks=True,
        ),
    )(ids_flat, table, w, b)
    return out[:N].reshape(B, S, H)


def kernel(ids, table, w, b):
    return _forward(ids, table, w, b, tm=512)


# speedup vs baseline: 2.2338x; 1.0543x over previous
import functools

import jax
import jax.numpy as jnp
from jax.experimental import pallas as pl
from jax.experimental.pallas import tpu as pltpu


def _issue_tile(ids_ref, table_hbm, emb_buf, sems, base, slot, tm):
    for r in range(tm):
        idx = ids_ref[base + r]
        pltpu.make_async_copy(
            table_hbm.at[pl.ds(idx, 1), :],
            emb_buf.at[slot, pl.ds(r, 1), :],
            sems.at[slot],
        ).start()


def _embed_project_body(ids_ref, table_hbm, w_ref, b_ref, out_ref,
                        emb_buf, sems, *, n_inner, tm):
    j = pl.program_id(0)

    @pl.when(j == 0)
    def _prime():
        _issue_tile(ids_ref, table_hbm, emb_buf, sems, j * tm, 0, tm)

    @pl.when(j + 1 < n_inner)
    def _prefetch():
        nxt = jax.lax.rem(j + 1, 2)
        _issue_tile(ids_ref, table_hbm, emb_buf, sems, (j + 1) * tm, nxt, tm)

    cur = jax.lax.rem(j, 2)
    pltpu.make_async_copy(
        table_hbm.at[pl.ds(0, tm), :], emb_buf.at[cur], sems.at[cur]
    ).wait()

    @pl.when(cur == 0)
    def _mm0():
        out_ref[...] = jnp.dot(emb_buf[0], w_ref[...],
                               preferred_element_type=jnp.float32) + b_ref[...]

    @pl.when(cur == 1)
    def _mm1():
        out_ref[...] = jnp.dot(emb_buf[1], w_ref[...],
                               preferred_element_type=jnp.float32) + b_ref[...]


@functools.partial(jax.jit, static_argnames=("tm",))
def _forward(ids, table, w, b, *, tm=256):
    B, S = ids.shape
    V, E = table.shape
    H = w.shape[1]
    N = B * S

    tm_eff = max(8, min(int(tm), ((N + 7) // 8) * 8))
    tm_eff = ((tm_eff + 7) // 8) * 8
    n_tiles = (N + tm_eff - 1) // tm_eff
    n_pad = n_tiles * tm_eff

    ids_flat = jnp.clip(ids.reshape(-1).astype(jnp.int32), 0, V - 1)
    if n_pad != N:
        ids_flat = jnp.pad(ids_flat, (0, n_pad - N))

    out = pl.pallas_call(
        functools.partial(_embed_project_body, n_inner=n_tiles, tm=tm_eff),
        out_shape=jax.ShapeDtypeStruct((n_pad, H), jnp.float32),
        grid_spec=pltpu.PrefetchScalarGridSpec(
            num_scalar_prefetch=1,
            grid=(n_tiles,),
            in_specs=[
                pl.BlockSpec(memory_space=pl.ANY),
                pl.BlockSpec((E, H), lambda j, ids: (0, 0)),
                pl.BlockSpec((1, H), lambda j, ids: (0, 0)),
            ],
            out_specs=pl.BlockSpec((tm_eff, H), lambda j, ids: (j, 0)),
            scratch_shapes=[
                pltpu.VMEM((2, tm_eff, E), table.dtype),
                pltpu.SemaphoreType.DMA((2,)),
            ],
        ),
        compiler_params=pltpu.CompilerParams(
            dimension_semantics=("arbitrary",),
            disable_bounds_checks=True,
        ),
    )(ids_flat, table, w, b)
    return out[:N].reshape(B, S, H)


def kernel(ids, table, w, b):
    return _forward(ids, table, w, b, tm=1024)


# speedup vs baseline: 2.2941x; 1.0270x over previous
import functools

import jax
import jax.numpy as jnp
from jax.experimental import pallas as pl
from jax.experimental.pallas import tpu as pltpu


def _issue_tile(ids_ref, table_hbm, emb_buf, sems, base, slot, tm):
    for r in range(tm):
        idx = ids_ref[base + r]
        pltpu.make_async_copy(
            table_hbm.at[pl.ds(idx, 1), :],
            emb_buf.at[slot, pl.ds(r, 1), :],
            sems.at[slot],
        ).start()


def _embed_project_body(ids_ref, table_hbm, w_ref, b_ref, out_ref,
                        emb_buf, sems, *, n_inner, tm):
    j = pl.program_id(0)

    @pl.when(j == 0)
    def _prime():
        _issue_tile(ids_ref, table_hbm, emb_buf, sems, j * tm, 0, tm)

    @pl.when(j + 1 < n_inner)
    def _prefetch():
        nxt = jax.lax.rem(j + 1, 2)
        _issue_tile(ids_ref, table_hbm, emb_buf, sems, (j + 1) * tm, nxt, tm)

    cur = jax.lax.rem(j, 2)
    pltpu.make_async_copy(
        table_hbm.at[pl.ds(0, tm), :], emb_buf.at[cur], sems.at[cur]
    ).wait()

    @pl.when(cur == 0)
    def _mm0():
        out_ref[...] = jnp.dot(emb_buf[0], w_ref[...],
                               preferred_element_type=jnp.float32) + b_ref[...]

    @pl.when(cur == 1)
    def _mm1():
        out_ref[...] = jnp.dot(emb_buf[1], w_ref[...],
                               preferred_element_type=jnp.float32) + b_ref[...]


@functools.partial(jax.jit, static_argnames=("tm",))
def _forward(ids, table, w, b, *, tm=256):
    B, S = ids.shape
    V, E = table.shape
    H = w.shape[1]
    N = B * S

    tm_eff = max(8, min(int(tm), ((N + 7) // 8) * 8))
    tm_eff = ((tm_eff + 7) // 8) * 8
    n_tiles = (N + tm_eff - 1) // tm_eff
    n_pad = n_tiles * tm_eff

    ids_flat = jnp.clip(ids.reshape(-1).astype(jnp.int32), 0, V - 1)
    if n_pad != N:
        ids_flat = jnp.pad(ids_flat, (0, n_pad - N))

    out = pl.pallas_call(
        functools.partial(_embed_project_body, n_inner=n_tiles, tm=tm_eff),
        out_shape=jax.ShapeDtypeStruct((n_pad, H), jnp.float32),
        grid_spec=pltpu.PrefetchScalarGridSpec(
            num_scalar_prefetch=1,
            grid=(n_tiles,),
            in_specs=[
                pl.BlockSpec(memory_space=pl.ANY),
                pl.BlockSpec((E, H), lambda j, ids: (0, 0)),
                pl.BlockSpec((1, H), lambda j, ids: (0, 0)),
            ],
            out_specs=pl.BlockSpec((tm_eff, H), lambda j, ids: (j, 0)),
            scratch_shapes=[
                pltpu.VMEM((2, tm_eff, E), table.dtype),
                pltpu.SemaphoreType.DMA((2,)),
            ],
        ),
        compiler_params=pltpu.CompilerParams(
            dimension_semantics=("arbitrary",),
            disable_bounds_checks=True,
        ),
    )(ids_flat, table, w, b)
    return out[:N].reshape(B, S, H)


def kernel(ids, table, w, b):
    return _forward(ids, table, w, b, tm=2048)
